# Initial kernel scaffold; baseline (speedup 1.0000x reference)
import jax
import jax.numpy as jnp
from jax.experimental import pallas as pl


def kernel(el_ids, elements_in_regions, values):
    raise NotImplementedError("write your pallas kernel here")


# speedup vs baseline: 6.1494x; 6.1494x over previous
import jax
import jax.numpy as jnp
from jax import lax
from jax.experimental import pallas as pl
from jax.experimental.pallas import tpu as pltpu
from jax.experimental.pallas import tpu_sc as plsc

NQ = 1_000_000
NE = 1_000_000
NR = 4
NSUB = 16

SB = 256
NSC = (NQ + SB - 1) // SB
SLAST = NQ - SB
KPT = 248
NROW = 7813
RLAST = NROW - 2

TRASH = 2 * NQ
TW = 4096
ZB = 800
ZSEG = 2 * NQ // NSUB
RZSEG = 62_496
DB = 512
DLAST = NQ - DB
DKPT = 124


def _sc_body(el_hbm, e2d_hbm, out0_hbm, out1_hbm, out2_hbm, out3_hbm,
             spm, d3, tb0, tb1, mb0, mb1, ones_v, zb_v, dm0, dm1, wv0, wv1,
             lsem, ssem, dsem, msem):
  c = lax.axis_index("c")
  s = lax.axis_index("s")
  e2f = e2d_hbm.at[jnp.int32(0)]

  def sem_drain(src, dst, sem, n=1):
    for _ in range(n):
      pltpu.make_async_copy(src, dst, sem).wait()

  def zfill(i, carry):
    zb_v[pl.ds(i * 16, 16)] = jnp.zeros((16,), jnp.int32)
    return carry
  lax.fori_loop(jnp.int32(0), jnp.int32(ZB // 16), zfill, jnp.int32(0))
  zbase = s * ZSEG
  def zfire(k, carry):
    pltpu.async_copy(zb_v, spm.at[pl.ds(zbase + k * ZB, ZB)], ssem)
    return carry
  lax.fori_loop(jnp.int32(0), jnp.int32(ZSEG // ZB), zfire, jnp.int32(0))
  pltpu.async_copy(zb_v.at[pl.ds(jnp.int32(0), ZSEG % ZB)],
                   spm.at[pl.ds(zbase + (ZSEG // ZB) * ZB, ZSEG % ZB)], ssem)
  for k in range(8):
    ones_v[pl.ds(16 * k, 16)] = jnp.full((16,), 1, jnp.int32)
  evens = [lax.iota(jnp.int32, 16) * 2 + 32 * g for g in range(16)]
  def zwait(k, carry):
    pltpu.make_async_copy(zb_v, spm.at[pl.ds(zbase + k * ZB, ZB)], ssem).wait()
    return carry
  lax.fori_loop(jnp.int32(0), jnp.int32(ZSEG // ZB), zwait, jnp.int32(0))
  pltpu.make_async_copy(zb_v.at[pl.ds(jnp.int32(0), ZSEG % ZB)],
                        spm.at[pl.ds(zbase, ZSEG % ZB)], ssem).wait()
  plsc.subcore_barrier()

  def soff(k):
    return jnp.minimum((s + 16 * k) * SB, SLAST).astype(jnp.int32)

  for b in range(2):
    o = soff(b)
    pltpu.async_copy(el_hbm.at[pl.ds(o, 128)], d3.at[jnp.int32(2 * b)], lsem)
    pltpu.async_copy(el_hbm.at[pl.ds(o + 128, 128)],
                     d3.at[jnp.int32(2 * b + 1)], lsem)

  def mstep(t, carry):
    for b in range(4):
      k = 4 * t + b
      sem_drain(el_hbm.at[pl.ds(jnp.int32(0), 128)], d3.at[jnp.int32(0)],
                lsem, 2)
      if b >= 2:
        sem_drain(ones_v, spm.at[pl.ds(jnp.int32(0), 128)], ssem, 2)
      else:
        @pl.when(t >= 1)
        def _():
          sem_drain(ones_v, spm.at[pl.ds(jnp.int32(0), 128)], ssem, 2)
      for rr in range(2):
        pltpu.async_copy(ones_v, spm.at[d3.at[jnp.int32(2 * b + rr)]], ssem, add=True)
      b2 = (b + 2) % 4
      o2 = soff(k + 2)
      pltpu.async_copy(el_hbm.at[pl.ds(o2, 128)], d3.at[jnp.int32(2 * b2)], lsem)
      pltpu.async_copy(el_hbm.at[pl.ds(o2 + 128, 128)],
                       d3.at[jnp.int32(2 * b2 + 1)], lsem)
    return carry

  lax.fori_loop(jnp.int32(0), jnp.int32(KPT // 4), mstep, jnp.int32(0))
  sem_drain(el_hbm.at[pl.ds(jnp.int32(0), 128)], d3.at[jnp.int32(0)], lsem, 4)
  sem_drain(ones_v, spm.at[pl.ds(jnp.int32(0), 128)], ssem, 4)
  plsc.subcore_barrier()

  for ph in range(2):
    if ph == 1:
      rzb = NQ + s * RZSEG
      def rzfire(k, carry):
        pltpu.async_copy(zb_v, spm.at[pl.ds(rzb + k * ZB, ZB)], ssem)
        return carry
      lax.fori_loop(jnp.int32(0), jnp.int32(RZSEG // ZB), rzfire, jnp.int32(0))
      pltpu.async_copy(zb_v.at[pl.ds(jnp.int32(0), RZSEG % ZB)],
                       spm.at[pl.ds(rzb + (RZSEG // ZB) * ZB, RZSEG % ZB)],
                       ssem)
      def rzwait(k, carry):
        pltpu.make_async_copy(zb_v, spm.at[pl.ds(rzb + k * ZB, ZB)],
                              ssem).wait()
        return carry
      lax.fori_loop(jnp.int32(0), jnp.int32(RZSEG // ZB), rzwait, jnp.int32(0))
      pltpu.make_async_copy(zb_v.at[pl.ds(jnp.int32(0), RZSEG % ZB)],
                            spm.at[pl.ds(rzb, RZSEG % ZB)], ssem).wait()
      @pl.when(s == 0)
      def _():
        pltpu.sync_copy(zb_v.at[pl.ds(jnp.int32(0), 64)],
                        spm.at[pl.ds(NQ + NSUB * RZSEG, 64)])
      plsc.subcore_barrier()

    treg = e2d_hbm.at[(2 * c + ph).astype(jnp.int32)]

    for b in range(2):
      o = soff(b)
      pltpu.async_copy(treg.at[pl.ds(2 * o, 2 * SB)], (tb0, tb1)[b], lsem)
      pltpu.async_copy(spm.at[pl.ds(o, SB)], (mb0, mb1)[b], msem)

    def tstep(t, carry):
      for b in range(4):
        k = 4 * t + b
        tbb = (tb0, tb1)[b % 2]
        mbb = (mb0, mb1)[b % 2]
        sem_drain(e2f.at[pl.ds(jnp.int32(0), 2 * SB)], tbb, lsem)
        sem_drain(e2f.at[pl.ds(jnp.int32(0), SB)], mbb, msem)
        if b >= 2:
          sem_drain(ones_v, spm.at[pl.ds(jnp.int32(0), 128)], ssem, 2)
        else:
          @pl.when(t >= 1)
          def _():
            sem_drain(ones_v, spm.at[pl.ds(jnp.int32(0), 128)], ssem, 2)
        for g in range(16):
          lows = plsc.load_gather(tbb, [evens[g]])
          hit = mbb[pl.ds(16 * g, 16)] > 0
          d3r = d3.at[jnp.int32(2 * b + (g // 8))]
          d3r[pl.ds((g % 8) * 16, 16)] = jnp.where(
              hit, lows + NQ, (lows & (TW - 1)) + TRASH)
        for rr in range(2):
          pltpu.async_copy(ones_v, spm.at[d3.at[jnp.int32(2 * b + rr)]], ssem, add=True)
        o2 = soff(k + 2)
        pltpu.async_copy(treg.at[pl.ds(2 * o2, 2 * SB)], tbb, lsem)
        pltpu.async_copy(spm.at[pl.ds(o2, SB)], mbb, msem)
      return carry

    lax.fori_loop(jnp.int32(0), jnp.int32(KPT // 4), tstep, jnp.int32(0))
    sem_drain(e2f.at[pl.ds(jnp.int32(0), 2 * SB)], tb0, lsem, 2)
    sem_drain(e2f.at[pl.ds(jnp.int32(0), SB)], mb0, msem, 2)
    sem_drain(ones_v, spm.at[pl.ds(jnp.int32(0), 128)], ssem, 4)
    plsc.subcore_barrier()

    outA = (out0_hbm, out1_hbm)[ph]
    outB = (out2_hbm, out3_hbm)[ph]

    def doff(k):
      return jnp.minimum((s + 16 * k) * DB, DLAST).astype(jnp.int32)

    for b in range(2):
      pltpu.async_copy(spm.at[pl.ds(NQ + doff(b), DB)], (dm0, dm1)[b], lsem)

    def dstep(t, carry):
      for b in range(2):
        k = 2 * t + b
        dmb = (dm0, dm1)[b]
        wvb = (wv0, wv1)[b]
        sem_drain(spm.at[pl.ds(jnp.int32(0), DB)], dmb, lsem)
        @pl.when(t >= 1)
        def _():
          sem_drain(wv0, outA.at[pl.ds(jnp.int32(0), DB)], dsem)
        def wstep(v, carry2):
          wvb[pl.ds(v * 16, 16)] = jnp.where(
              dmb[pl.ds(v * 16, 16)] > 0, jnp.int32(1), jnp.int32(0))
          return carry2
        lax.fori_loop(jnp.int32(0), jnp.int32(DB // 16), wstep, jnp.int32(0))
        o = doff(k)
        @pl.when(c == 0)
        def _():
          pltpu.async_copy(wvb, outA.at[pl.ds(o, DB)], dsem)
        @pl.when(c == 1)
        def _():
          pltpu.async_copy(wvb, outB.at[pl.ds(o, DB)], dsem)
        pltpu.async_copy(spm.at[pl.ds(NQ + doff(k + 2), DB)], dmb, lsem)
      return carry

    lax.fori_loop(jnp.int32(0), jnp.int32(DKPT // 2), dstep, jnp.int32(0))
    sem_drain(spm.at[pl.ds(jnp.int32(0), DB)], dm0, lsem, 2)
    sem_drain(wv0, outA.at[pl.ds(jnp.int32(0), DB)], dsem, 2)
    plsc.subcore_barrier()


_sc_cache = []


def _sc_call(el, e2d):
  if not _sc_cache:
    _sc_cache.append(pl.kernel(
        _sc_body,
        out_type=tuple(jax.ShapeDtypeStruct((NQ,), jnp.int32)
                       for _ in range(NR)),
        mesh=plsc.VectorSubcoreMesh(core_axis_name="c", subcore_axis_name="s",
                                    num_cores=2, num_subcores=NSUB),
        compiler_params=pltpu.CompilerParams(needs_layout_passes=False),
        scratch_types=[
            pltpu.VMEM_SHARED((2 * NQ + TW,), jnp.int32),
            pltpu.VMEM((8, 128), jnp.int32),
            pltpu.VMEM((2 * SB,), jnp.int32),
            pltpu.VMEM((2 * SB,), jnp.int32),
            pltpu.VMEM((SB,), jnp.int32),
            pltpu.VMEM((SB,), jnp.int32),
            pltpu.VMEM((128,), jnp.int32),
            pltpu.VMEM((ZB,), jnp.int32),
            pltpu.VMEM((DB,), jnp.int32),
            pltpu.VMEM((DB,), jnp.int32),
            pltpu.VMEM((DB,), jnp.int32),
            pltpu.VMEM((DB,), jnp.int32),
            pltpu.SemaphoreType.DMA,
            pltpu.SemaphoreType.DMA,
            pltpu.SemaphoreType.DMA,
            pltpu.SemaphoreType.DMA,
        ],
    ))
  return _sc_cache[0](el, e2d)


def _combine_body(w0_ref, w1_ref, w2_ref, w3_ref, vals_ref, out_ref):
  v3 = vals_ref[0, 3]
  v2 = vals_ref[0, 2]
  v1 = vals_ref[0, 1]
  v0 = vals_ref[0, 0]
  z = jnp.float32(0.0)
  out_ref[...] = jnp.where(
      w3_ref[...] > 0, v3,
      jnp.where(w2_ref[...] > 0, v2,
                jnp.where(w1_ref[...] > 0, v1,
                          jnp.where(w0_ref[...] > 0, v0, z))))


_ROWS, _COLS = 1000, 1000
_BR = 8

_combine = pl.pallas_call(
    _combine_body,
    grid=(_ROWS // _BR,),
    in_specs=[pl.BlockSpec((_BR, _COLS),
                           lambda i: (jnp.asarray(i, jnp.int32), jnp.int32(0)))] * NR
    + [pl.BlockSpec((1, NR), lambda i: (jnp.int32(0), jnp.int32(0)),
                    memory_space=pltpu.SMEM)],
    out_specs=pl.BlockSpec((_BR, _COLS),
                           lambda i: (jnp.asarray(i, jnp.int32), jnp.int32(0))),
    out_shape=jax.ShapeDtypeStruct((_ROWS, _COLS), jnp.float32),
)


def kernel(el_ids, elements_in_regions, values):
  el = el_ids.astype(jnp.int32)
  e2d = lax.bitcast_convert_type(elements_in_regions, jnp.int32)
  e2d = e2d.reshape(NR, 2 * NE)
  w0, w1, w2, w3 = _sc_call(el, e2d)
  pv = _combine(w0.reshape(_ROWS, _COLS), w1.reshape(_ROWS, _COLS),
                w2.reshape(_ROWS, _COLS), w3.reshape(_ROWS, _COLS),
                values.reshape(1, NR).astype(jnp.float32))
  return pv.reshape(NQ)


# speedup vs baseline: 40.3749x; 6.5657x over previous
import jax
import jax.numpy as jnp
from jax import lax
from jax.experimental import pallas as pl
from jax.experimental.pallas import tpu as pltpu
from jax.experimental.pallas import tpu_sc as plsc

NQ = 1_000_000
NE = 1_000_000
NR = 4
NSUB = 16

SB = 320
NSC = NQ // SB
SLAST = NQ - SB
KPT = 196

TRASH = 2 * NQ
TW = 4096
ZB = 400
ZSEG = 2 * NQ // NSUB
RZSEG = 62_496
DB = 400
NDB = NQ // DB
DKPT = 158


def _sc_body(el_hbm, t0_hbm, t1_hbm, t2_hbm, t3_hbm,
             out0_hbm, out1_hbm, out2_hbm, out3_hbm,
             spm, d3, d64, tb0, tb1, mb0, mb1, ones_v, zb_v, dm0, dm1, wv0, wv1,
             lsem, ssem, dsem, msem):
  c = lax.axis_index("c")
  s = lax.axis_index("s")
  e2f = t0_hbm

  def sem_drain(src, dst, sem, n=1):
    for _ in range(n):
      pltpu.make_async_copy(src, dst, sem).wait()

  def zfill(i, carry):
    zb_v[pl.ds(i * 16, 16)] = jnp.zeros((16,), jnp.int32)
    return carry
  lax.fori_loop(jnp.int32(0), jnp.int32(ZB // 16), zfill, jnp.int32(0))
  zbase = s * ZSEG
  def zfire(k, carry):
    pltpu.async_copy(zb_v, spm.at[pl.ds(zbase + k * ZB, ZB)], ssem)
    return carry
  lax.fori_loop(jnp.int32(0), jnp.int32(ZSEG // ZB), zfire, jnp.int32(0))
  pltpu.async_copy(zb_v.at[pl.ds(jnp.int32(0), ZSEG % ZB)],
                   spm.at[pl.ds(zbase + (ZSEG // ZB) * ZB, ZSEG % ZB)], ssem)
  for k in range(8):
    ones_v[pl.ds(16 * k, 16)] = jnp.full((16,), 1, jnp.int32)
  def zwait(k, carry):
    pltpu.make_async_copy(zb_v, spm.at[pl.ds(zbase + k * ZB, ZB)], ssem).wait()
    return carry
  lax.fori_loop(jnp.int32(0), jnp.int32(ZSEG // ZB), zwait, jnp.int32(0))
  pltpu.make_async_copy(zb_v.at[pl.ds(jnp.int32(0), ZSEG % ZB)],
                        spm.at[pl.ds(zbase, ZSEG % ZB)], ssem).wait()
  plsc.subcore_barrier()

  def soff(k):
    return jnp.minimum((s + 16 * k) * SB, SLAST).astype(jnp.int32)

  def mload(o, b):
    pltpu.async_copy(el_hbm.at[pl.ds(o, 128)], d3.at[jnp.int32(2 * b)], lsem)
    pltpu.async_copy(el_hbm.at[pl.ds(o + 128, 128)],
                     d3.at[jnp.int32(2 * b + 1)], lsem)
    pltpu.async_copy(el_hbm.at[pl.ds(o + 256, 64)], d64.at[jnp.int32(b)], lsem)

  def mdrain(sem, n=1):
    for _ in range(n):
      sem_drain(el_hbm.at[pl.ds(jnp.int32(0), 128)], d3.at[jnp.int32(0)],
                sem, 2)
      sem_drain(el_hbm.at[pl.ds(jnp.int32(0), 64)], d64.at[jnp.int32(0)], sem)

  for b in range(2):
    mload(soff(b), b)

  def mstep(t, carry):
    for b in range(4):
      k = 4 * t + b
      mdrain(lsem)
      if b >= 2:
        mdrain(ssem)
      else:
        @pl.when(t >= 1)
        def _():
          mdrain(ssem)
      for rr in range(2):
        pltpu.async_copy(ones_v, spm.at[d3.at[jnp.int32(2 * b + rr)]], ssem, add=True)
      pltpu.async_copy(ones_v.at[pl.ds(jnp.int32(0), 64)],
                       spm.at[d64.at[jnp.int32(b)]], ssem, add=True)
      mload(soff(k + 2), (b + 2) % 4)
    return carry

  lax.fori_loop(jnp.int32(0), jnp.int32(KPT // 4), mstep, jnp.int32(0))
  mdrain(lsem, 2)
  mdrain(ssem, 2)
  plsc.subcore_barrier()

  for ph in range(2):
    if ph == 1:
      rzb = NQ + s * RZSEG
      def rzfire(k, carry):
        pltpu.async_copy(zb_v, spm.at[pl.ds(rzb + k * ZB, ZB)], ssem)
        return carry
      lax.fori_loop(jnp.int32(0), jnp.int32(RZSEG // ZB), rzfire, jnp.int32(0))
      pltpu.async_copy(zb_v.at[pl.ds(jnp.int32(0), RZSEG % ZB)],
                       spm.at[pl.ds(rzb + (RZSEG // ZB) * ZB, RZSEG % ZB)],
                       ssem)
      def rzwait(k, carry):
        pltpu.make_async_copy(zb_v, spm.at[pl.ds(rzb + k * ZB, ZB)],
                              ssem).wait()
        return carry
      lax.fori_loop(jnp.int32(0), jnp.int32(RZSEG // ZB), rzwait, jnp.int32(0))
      pltpu.make_async_copy(zb_v.at[pl.ds(jnp.int32(0), RZSEG % ZB)],
                            spm.at[pl.ds(rzb, RZSEG % ZB)], ssem).wait()
      @pl.when(s == 0)
      def _():
        pltpu.sync_copy(zb_v.at[pl.ds(jnp.int32(0), 64)],
                        spm.at[pl.ds(NQ + NSUB * RZSEG, 64)])
      plsc.subcore_barrier()

    tA = (t0_hbm, t1_hbm)[ph]
    tB = (t2_hbm, t3_hbm)[ph]

    def tload(o, tbb, mbb):
      @pl.when(c == 0)
      def _():
        pltpu.async_copy(tA.at[pl.ds(o, SB)], tbb, lsem)
      @pl.when(c == 1)
      def _():
        pltpu.async_copy(tB.at[pl.ds(o, SB)], tbb, lsem)
      pltpu.async_copy(spm.at[pl.ds(o, SB)], mbb, msem)

    for b in range(2):
      tload(soff(b), (tb0, tb1)[b], (mb0, mb1)[b])

    def tstep(t, carry):
      for b in range(4):
        k = 4 * t + b
        tbb = (tb0, tb1)[b % 2]
        mbb = (mb0, mb1)[b % 2]
        sem_drain(e2f.at[pl.ds(c * 0, SB)], tbb, lsem)
        sem_drain(e2f.at[pl.ds(c * 0, SB)], mbb, msem)
        if b >= 2:
          mdrain(ssem)
        else:
          @pl.when(t >= 1)
          def _():
            mdrain(ssem)
        for g in range(20):
          lows = tbb[pl.ds(16 * g, 16)]
          hit = mbb[pl.ds(16 * g, 16)] > 0
          d = jnp.where(hit, lows + NQ, (lows & (TW - 1)) + TRASH)
          if g < 16:
            d3r = d3.at[jnp.int32(2 * b + (g // 8))]
            d3r[pl.ds((g % 8) * 16, 16)] = d
          else:
            d64r = d64.at[jnp.int32(b)]
            d64r[pl.ds((g - 16) * 16, 16)] = d
        for rr in range(2):
          pltpu.async_copy(ones_v, spm.at[d3.at[jnp.int32(2 * b + rr)]], ssem, add=True)
        pltpu.async_copy(ones_v.at[pl.ds(jnp.int32(0), 64)],
                         spm.at[d64.at[jnp.int32(b)]], ssem, add=True)
        tload(soff(k + 2), tbb, mbb)
      return carry

    lax.fori_loop(jnp.int32(0), jnp.int32(KPT // 4), tstep, jnp.int32(0))
    sem_drain(e2f.at[pl.ds(c * 0, SB)], tb0, lsem, 2)
    sem_drain(e2f.at[pl.ds(c * 0, SB)], mb0, msem, 2)
    mdrain(ssem, 2)
    plsc.subcore_barrier()

    outA = (out0_hbm, out1_hbm)[ph]
    outB = (out2_hbm, out3_hbm)[ph]

    def doff(k):
      return (jnp.minimum(s + 16 * k, NDB - 1) * DB).astype(jnp.int32)

    for b in range(2):
      pltpu.async_copy(spm.at[pl.ds(NQ + doff(b), DB)], (dm0, dm1)[b], lsem)

    def dstep(t, carry):
      for b in range(2):
        k = 2 * t + b
        dmb = (dm0, dm1)[b]
        wvb = (wv0, wv1)[b]
        sem_drain(spm.at[pl.ds(jnp.int32(0), DB)], dmb, lsem)
        @pl.when(t >= 1)
        def _():
          sem_drain(wv0, outA.at[pl.ds(c * 0, DB)], dsem)
        def wstep(v, carry2):
          wvb[pl.ds(v * 16, 16)] = jnp.where(
              dmb[pl.ds(v * 16, 16)] > 0, jnp.int32(1), jnp.int32(0))
          return carry2
        lax.fori_loop(jnp.int32(0), jnp.int32(DB // 16), wstep, jnp.int32(0))
        o = doff(k)
        @pl.when(c == 0)
        def _():
          pltpu.async_copy(wvb, outA.at[pl.ds(o, DB)], dsem)
        @pl.when(c == 1)
        def _():
          pltpu.async_copy(wvb, outB.at[pl.ds(o, DB)], dsem)
        pltpu.async_copy(spm.at[pl.ds(NQ + doff(k + 2), DB)], dmb, lsem)
      return carry

    lax.fori_loop(jnp.int32(0), jnp.int32(DKPT // 2), dstep, jnp.int32(0))
    sem_drain(spm.at[pl.ds(jnp.int32(0), DB)], dm0, lsem, 2)
    sem_drain(wv0, outA.at[pl.ds(c * 0, DB)], dsem, 2)
    plsc.subcore_barrier()


_sc_cache = []


def _sc_call(el, t0, t1, t2, t3):
  if not _sc_cache:
    _sc_cache.append(pl.kernel(
        _sc_body,
        out_type=tuple(jax.ShapeDtypeStruct((NQ,), jnp.int32)
                       for _ in range(NR)),
        mesh=plsc.VectorSubcoreMesh(core_axis_name="c", subcore_axis_name="s",
                                    num_cores=2, num_subcores=NSUB),
        compiler_params=pltpu.CompilerParams(needs_layout_passes=False),
        scratch_types=[
            pltpu.VMEM_SHARED((2 * NQ + TW,), jnp.int32),
            pltpu.VMEM((8, 128), jnp.int32),
            pltpu.VMEM((4, 64), jnp.int32),
            pltpu.VMEM((SB,), jnp.int32),
            pltpu.VMEM((SB,), jnp.int32),
            pltpu.VMEM((SB,), jnp.int32),
            pltpu.VMEM((SB,), jnp.int32),
            pltpu.VMEM((128,), jnp.int32),
            pltpu.VMEM((ZB,), jnp.int32),
            pltpu.VMEM((DB,), jnp.int32),
            pltpu.VMEM((DB,), jnp.int32),
            pltpu.VMEM((DB,), jnp.int32),
            pltpu.VMEM((DB,), jnp.int32),
            pltpu.SemaphoreType.DMA,
            pltpu.SemaphoreType.DMA,
            pltpu.SemaphoreType.DMA,
            pltpu.SemaphoreType.DMA,
        ],
    ))
  return _sc_cache[0](el, t0, t1, t2, t3)


def _combine_body(w0_ref, w1_ref, w2_ref, w3_ref, vals_ref, out_ref):
  v3 = vals_ref[0, 3]
  v2 = vals_ref[0, 2]
  v1 = vals_ref[0, 1]
  v0 = vals_ref[0, 0]
  z = jnp.float32(0.0)
  out_ref[...] = jnp.where(
      w3_ref[...] > 0, v3,
      jnp.where(w2_ref[...] > 0, v2,
                jnp.where(w1_ref[...] > 0, v1,
                          jnp.where(w0_ref[...] > 0, v0, z))))


_CB = 8_192

_combine = pl.pallas_call(
    _combine_body,
    grid=((NQ + _CB - 1) // _CB,),
    in_specs=[pl.BlockSpec((_CB,), lambda i: (jnp.asarray(i, jnp.int32),))] * NR
    + [pl.BlockSpec((1, NR), lambda i: (jnp.int32(0), jnp.int32(0)),
                    memory_space=pltpu.SMEM)],
    out_specs=pl.BlockSpec((_CB,), lambda i: (jnp.asarray(i, jnp.int32),)),
    out_shape=jax.ShapeDtypeStruct((NQ,), jnp.float32),
)


def kernel(el_ids, elements_in_regions, values):
  el = el_ids.astype(jnp.int32)
  t0, t1, t2, t3 = (elements_in_regions[r].astype(jnp.int32)
                    for r in range(NR))
  w0, w1, w2, w3 = _sc_call(el, t0, t1, t2, t3)
  return _combine(w0, w1, w2, w3,
                  values.reshape(1, NR).astype(jnp.float32))


# speedup vs baseline: 40.6080x; 1.0058x over previous
import jax
import jax.numpy as jnp
from jax import lax
from jax.experimental import pallas as pl
from jax.experimental.pallas import tpu as pltpu
from jax.experimental.pallas import tpu_sc as plsc

NQ = 1_000_000
NE = 1_000_000
NR = 4
NSUB = 16

SB = 320
NSC = NQ // SB
SLAST = NQ - SB
KPT = 196

TRASH = 2 * NQ
TW = 4096
ZB = 320
ZSEG = 2 * NQ // NSUB
BIG = float(1 << 24)
DB = 400
NDB = NQ // DB
DKPT = 158


def _sc_body(el_hbm, t0_hbm, t1_hbm, t2_hbm, t3_hbm,
             out0_hbm, out1_hbm, out2_hbm, out3_hbm,
             spm, d3, d64, tb0, tb1, mb0, mb1, ones_v, bigs_v, zb_v, dm0, dm1, wv0, wv1,
             lsem, ssem, dsem, msem):
  c = lax.axis_index("c")
  s = lax.axis_index("s")
  e2f = t0_hbm

  def sem_drain(src, dst, sem, n=1):
    for _ in range(n):
      pltpu.make_async_copy(src, dst, sem).wait()

  def zfill(i, carry):
    zb_v[pl.ds(i * 16, 16)] = jnp.zeros((16,), jnp.float32)
    return carry
  lax.fori_loop(jnp.int32(0), jnp.int32(ZB // 16), zfill, jnp.int32(0))
  zbase = s * ZSEG
  def zfire(k, carry):
    pltpu.async_copy(zb_v, spm.at[pl.ds(zbase + k * ZB, ZB)], ssem)
    return carry
  lax.fori_loop(jnp.int32(0), jnp.int32(ZSEG // ZB), zfire, jnp.int32(0))
  pltpu.async_copy(zb_v.at[pl.ds(jnp.int32(0), ZSEG % ZB)],
                   spm.at[pl.ds(zbase + (ZSEG // ZB) * ZB, ZSEG % ZB)], ssem)
  for k in range(8):
    ones_v[pl.ds(16 * k, 16)] = jnp.full((16,), 1.0, jnp.float32)
    bigs_v[pl.ds(16 * k, 16)] = jnp.full((16,), BIG, jnp.float32)
  def zwait(k, carry):
    pltpu.make_async_copy(zb_v, spm.at[pl.ds(zbase + k * ZB, ZB)], ssem).wait()
    return carry
  lax.fori_loop(jnp.int32(0), jnp.int32(ZSEG // ZB), zwait, jnp.int32(0))
  pltpu.make_async_copy(zb_v.at[pl.ds(jnp.int32(0), ZSEG % ZB)],
                        spm.at[pl.ds(zbase, ZSEG % ZB)], ssem).wait()
  plsc.subcore_barrier()

  def soff(k):
    return jnp.minimum((s + 16 * k) * SB, SLAST).astype(jnp.int32)

  def mload(o, b):
    pltpu.async_copy(el_hbm.at[pl.ds(o, 128)], d3.at[jnp.int32(2 * b)], lsem)
    pltpu.async_copy(el_hbm.at[pl.ds(o + 128, 128)],
                     d3.at[jnp.int32(2 * b + 1)], lsem)
    pltpu.async_copy(el_hbm.at[pl.ds(o + 256, 64)], d64.at[jnp.int32(b)], lsem)

  def mdrain(sem, n=1):
    for _ in range(n):
      sem_drain(el_hbm.at[pl.ds(jnp.int32(0), 128)], d3.at[jnp.int32(0)],
                sem, 2)
      sem_drain(el_hbm.at[pl.ds(jnp.int32(0), 64)], d64.at[jnp.int32(0)], sem)

  for b in range(2):
    mload(soff(b), b)

  def mstep(t, carry):
    for b in range(4):
      k = 4 * t + b
      mdrain(lsem)
      if b >= 2:
        mdrain(ssem)
      else:
        @pl.when(t >= 1)
        def _():
          mdrain(ssem)
      for rr in range(2):
        pltpu.async_copy(ones_v, spm.at[d3.at[jnp.int32(2 * b + rr)]], ssem, add=True)
      pltpu.async_copy(ones_v.at[pl.ds(jnp.int32(0), 64)],
                       spm.at[d64.at[jnp.int32(b)]], ssem, add=True)
      mload(soff(k + 2), (b + 2) % 4)
    return carry

  lax.fori_loop(jnp.int32(0), jnp.int32(KPT // 4), mstep, jnp.int32(0))
  mdrain(lsem, 2)
  mdrain(ssem, 2)
  plsc.subcore_barrier()

  for ph in range(2):
    inc_v = (ones_v, bigs_v)[ph]
    tA = (t0_hbm, t1_hbm)[ph]
    tB = (t2_hbm, t3_hbm)[ph]

    def tload(o, tbb, mbb):
      @pl.when(c == 0)
      def _():
        pltpu.async_copy(tA.at[pl.ds(o, SB)], tbb, lsem)
      @pl.when(c == 1)
      def _():
        pltpu.async_copy(tB.at[pl.ds(o, SB)], tbb, lsem)
      pltpu.async_copy(spm.at[pl.ds(o, SB)], mbb, msem)

    for b in range(2):
      tload(soff(b), (tb0, tb1)[b], (mb0, mb1)[b])

    def tstep(t, carry):
      for b in range(4):
        k = 4 * t + b
        tbb = (tb0, tb1)[b % 2]
        mbb = (mb0, mb1)[b % 2]
        sem_drain(e2f.at[pl.ds(c * 0, SB)], tbb, lsem)
        sem_drain(e2f.at[pl.ds(c * 0, SB)], mbb, msem)
        if b >= 2:
          mdrain(ssem)
        else:
          @pl.when(t >= 1)
          def _():
            mdrain(ssem)
        for g in range(20):
          lows = tbb[pl.ds(16 * g, 16)]
          hit = mbb[pl.ds(16 * g, 16)] > 0.0
          d = jnp.where(hit, lows + NQ, (lows & (TW - 1)) + TRASH)
          if g < 16:
            d3r = d3.at[jnp.int32(2 * b + (g // 8))]
            d3r[pl.ds((g % 8) * 16, 16)] = d
          else:
            d64r = d64.at[jnp.int32(b)]
            d64r[pl.ds((g - 16) * 16, 16)] = d
        for rr in range(2):
          pltpu.async_copy(inc_v, spm.at[d3.at[jnp.int32(2 * b + rr)]], ssem, add=True)
        pltpu.async_copy(inc_v.at[pl.ds(jnp.int32(0), 64)],
                         spm.at[d64.at[jnp.int32(b)]], ssem, add=True)
        tload(soff(k + 2), tbb, mbb)
      return carry

    lax.fori_loop(jnp.int32(0), jnp.int32(KPT // 4), tstep, jnp.int32(0))
    sem_drain(e2f.at[pl.ds(c * 0, SB)], tb0, lsem, 2)
    sem_drain(e2f.at[pl.ds(c * 0, SB)], mb0, msem, 2)
    mdrain(ssem, 2)
    plsc.subcore_barrier()

    outA = (out0_hbm, out1_hbm)[ph]
    outB = (out2_hbm, out3_hbm)[ph]

    def doff(k):
      return (jnp.minimum(s + 16 * k, NDB - 1) * DB).astype(jnp.int32)

    for b in range(2):
      pltpu.async_copy(spm.at[pl.ds(NQ + doff(b), DB)], (dm0, dm1)[b], lsem)

    def dstep(t, carry):
      for b in range(2):
        k = 2 * t + b
        dmb = (dm0, dm1)[b]
        wvb = (wv0, wv1)[b]
        sem_drain(spm.at[pl.ds(jnp.int32(0), DB)], dmb, lsem)
        @pl.when(t >= 1)
        def _():
          sem_drain(wv0, outA.at[pl.ds(c * 0, DB)], dsem)
        th = jnp.float32((0.5, BIG - 1.0)[ph])
        def wstep(v, carry2):
          wvb[pl.ds(v * 16, 16)] = jnp.where(
              dmb[pl.ds(v * 16, 16)] > th, jnp.int32(1), jnp.int32(0))
          return carry2
        lax.fori_loop(jnp.int32(0), jnp.int32(DB // 16), wstep, jnp.int32(0))
        o = doff(k)
        @pl.when(c == 0)
        def _():
          pltpu.async_copy(wvb, outA.at[pl.ds(o, DB)], dsem)
        @pl.when(c == 1)
        def _():
          pltpu.async_copy(wvb, outB.at[pl.ds(o, DB)], dsem)
        pltpu.async_copy(spm.at[pl.ds(NQ + doff(k + 2), DB)], dmb, lsem)
      return carry

    lax.fori_loop(jnp.int32(0), jnp.int32(DKPT // 2), dstep, jnp.int32(0))
    sem_drain(spm.at[pl.ds(jnp.int32(0), DB)], dm0, lsem, 2)
    sem_drain(wv0, outA.at[pl.ds(c * 0, DB)], dsem, 2)
    plsc.subcore_barrier()


_sc_cache = []


def _sc_call(el, t0, t1, t2, t3):
  if not _sc_cache:
    _sc_cache.append(pl.kernel(
        _sc_body,
        out_type=tuple(jax.ShapeDtypeStruct((NQ,), jnp.int32)
                       for _ in range(NR)),
        mesh=plsc.VectorSubcoreMesh(core_axis_name="c", subcore_axis_name="s",
                                    num_cores=2, num_subcores=NSUB),
        compiler_params=pltpu.CompilerParams(needs_layout_passes=False),
        scratch_types=[
            pltpu.VMEM_SHARED((2 * NQ + TW,), jnp.float32),
            pltpu.VMEM((8, 128), jnp.int32),
            pltpu.VMEM((4, 64), jnp.int32),
            pltpu.VMEM((SB,), jnp.int32),
            pltpu.VMEM((SB,), jnp.int32),
            pltpu.VMEM((SB,), jnp.float32),
            pltpu.VMEM((SB,), jnp.float32),
            pltpu.VMEM((128,), jnp.float32),
            pltpu.VMEM((128,), jnp.float32),
            pltpu.VMEM((ZB,), jnp.float32),
            pltpu.VMEM((DB,), jnp.float32),
            pltpu.VMEM((DB,), jnp.float32),
            pltpu.VMEM((DB,), jnp.int32),
            pltpu.VMEM((DB,), jnp.int32),
            pltpu.SemaphoreType.DMA,
            pltpu.SemaphoreType.DMA,
            pltpu.SemaphoreType.DMA,
            pltpu.SemaphoreType.DMA,
        ],
    ))
  return _sc_cache[0](el, t0, t1, t2, t3)


def _combine_body(w0_ref, w1_ref, w2_ref, w3_ref, vals_ref, out_ref):
  v3 = vals_ref[0, 3]
  v2 = vals_ref[0, 2]
  v1 = vals_ref[0, 1]
  v0 = vals_ref[0, 0]
  z = jnp.float32(0.0)
  out_ref[...] = jnp.where(
      w3_ref[...] > 0, v3,
      jnp.where(w2_ref[...] > 0, v2,
                jnp.where(w1_ref[...] > 0, v1,
                          jnp.where(w0_ref[...] > 0, v0, z))))


_CB = 8_192

_combine = pl.pallas_call(
    _combine_body,
    grid=((NQ + _CB - 1) // _CB,),
    in_specs=[pl.BlockSpec((_CB,), lambda i: (jnp.asarray(i, jnp.int32),))] * NR
    + [pl.BlockSpec((1, NR), lambda i: (jnp.int32(0), jnp.int32(0)),
                    memory_space=pltpu.SMEM)],
    out_specs=pl.BlockSpec((_CB,), lambda i: (jnp.asarray(i, jnp.int32),)),
    out_shape=jax.ShapeDtypeStruct((NQ,), jnp.float32),
)


def kernel(el_ids, elements_in_regions, values):
  el = el_ids.astype(jnp.int32)
  t0, t1, t2, t3 = (elements_in_regions[r].astype(jnp.int32)
                    for r in range(NR))
  w0, w1, w2, w3 = _sc_call(el, t0, t1, t2, t3)
  return _combine(w0, w1, w2, w3,
                  values.reshape(1, NR).astype(jnp.float32))
